# baseline (speedup 1.0000x reference)
import jax
import jax.numpy as jnp
from jax.experimental import pallas as pl
from jax.experimental.pallas import tpu as pltpu

_ALPHA = 0.2
_PACK = 4
_BB = 32


def _lrelu(v):
    return jnp.maximum(v, _ALPHA * v)


def _mp_kernel(x_ref, x4_ref, feW0_ref, feb0_ref, feW1_ref, feb1_ref,
               fnW0_ref, fnb0_ref, fnW1_ref, fnb1_ref, out_ref,
               w0a4_s, w1bd_s, p4_s, na4_s, nb4_s, nw14_s):
    k = _PACK
    n, d = x_ref.shape[1], x_ref.shape[2]
    m = n // k
    f1 = feW1_ref.shape[0]
    f2 = feW1_ref.shape[1]
    fn = fnW0_ref.shape[1]
    fo = fnW1_ref.shape[1]

    @pl.when(pl.program_id(0) == 0)
    def _prep():
        w0a4_s[...] = jnp.zeros_like(w0a4_s)
        w1bd_s[...] = jnp.zeros_like(w1bd_s)
        p4_s[...] = jnp.zeros_like(p4_s)
        na4_s[...] = jnp.zeros_like(na4_s)
        nb4_s[...] = jnp.zeros_like(nb4_s)
        nw14_s[...] = jnp.zeros_like(nw14_s)
        for i in range(k):
            w0a4_s[i * d:(i + 1) * d, i * f1:(i + 1) * f1] = feW0_ref[0:d]
            w1bd_s[i * f1:(i + 1) * f1, i * f2:(i + 1) * f2] = feW1_ref[...]
            p4_s[i:i + 1, i * f1:(i + 1) * f1] = feW0_ref[2 * d:2 * d + 1]
            na4_s[i * f2:(i + 1) * f2, i * fn:(i + 1) * fn] = fnW0_ref[0:f2]
            nb4_s[i * d:(i + 1) * d, i * fn:(i + 1) * fn] = fnW0_ref[f2:]
            nw14_s[i * fn:(i + 1) * fn, i * fo:(i + 1) * fo] = fnW1_ref[...]

    w0b = feW0_ref[d:2 * d]
    b04 = jnp.tile(feb0_ref[...], (1, k))
    b14 = jnp.tile(feb1_ref[...], (1, k))
    nb04 = jnp.tile(fnb0_ref[...], (1, k))
    nb14 = jnp.tile(fnb1_ref[...], (1, k))

    for s in range(x_ref.shape[0]):
        x = x_ref[s]

        u4 = jnp.dot(x4_ref[s], w0a4_s[...],
                     preferred_element_type=jnp.float32)
        v = jnp.dot(x, w0b, preferred_element_type=jnp.float32)
        v4 = jnp.tile(v, (1, k)) + b04

        xx = x * x
        g = jax.lax.dot_general(x, x, (((1,), (1,)), ((), ())),
                                preferred_element_type=jnp.float32)
        sq_col = jnp.sum(xx, axis=1, keepdims=True)
        ones_row = jnp.ones((1, d), jnp.float32)
        sq_row = jax.lax.dot_general(ones_row, xx, (((1,), (1,)), ((), ())),
                                     preferred_element_type=jnp.float32)
        dist = jnp.sqrt(jnp.maximum(sq_col + sq_row - 2.0 * g, 0.0))

        dist4 = jnp.transpose(dist.reshape(m, k, n),
                              (0, 2, 1)).reshape(m * n, k)
        dterm = jnp.dot(dist4, p4_s[...],
                        preferred_element_type=jnp.float32)

        e = u4[:, None, :] + v4[None, :, :] + dterm.reshape(m, n, k * d)
        e = _lrelu(e)

        e2 = jnp.dot(e.reshape(m * n, k * d), w1bd_s[...],
                     preferred_element_type=jnp.float32) + b14
        e2 = _lrelu(e2)

        agg4 = jnp.sum(e2.reshape(m, n, -1), axis=1)

        h = jnp.dot(agg4, na4_s[...], preferred_element_type=jnp.float32)
        h = h + jnp.dot(x4_ref[s], nb4_s[...],
                        preferred_element_type=jnp.float32)
        h = _lrelu(h + nb04)
        out_ref[s] = jnp.dot(h, nw14_s[...],
                             preferred_element_type=jnp.float32) + nb14


def kernel(x, fe_W0, fe_b0, fe_W1, fe_b1, fn_W0, fn_b0, fn_W1, fn_b1):
    B, N, D = x.shape
    F1 = fe_W0.shape[1]
    F2 = fe_W1.shape[1]
    FN = fn_W0.shape[1]
    FO = fn_W1.shape[1]
    k = _PACK
    m = N // k
    bb = _BB

    full = lambda shape: pl.BlockSpec(shape, lambda b: (0,) * len(shape))

    out4 = pl.pallas_call(
        _mp_kernel,
        grid=(B // bb,),
        in_specs=[
            pl.BlockSpec((bb, N, D), lambda b: (b, 0, 0)),
            pl.BlockSpec((bb, m, k * D), lambda b: (b, 0, 0)),
            full(fe_W0.shape), full((1, F1)), full(fe_W1.shape), full((1, F2)),
            full(fn_W0.shape), full((1, FN)), full(fn_W1.shape), full((1, FO)),
        ],
        out_specs=pl.BlockSpec((bb, m, k * FO), lambda b: (b, 0, 0)),
        out_shape=jax.ShapeDtypeStruct((B, m, k * FO), jnp.float32),
        scratch_shapes=[
            pltpu.VMEM((k * D, k * F1), jnp.float32),
            pltpu.VMEM((k * F1, k * F2), jnp.float32),
            pltpu.VMEM((k, k * F1), jnp.float32),
            pltpu.VMEM((k * F2, k * FN), jnp.float32),
            pltpu.VMEM((k * D, k * FN), jnp.float32),
            pltpu.VMEM((k * FN, k * FO), jnp.float32),
        ],
    )(x, x.reshape(B, m, k * D), fe_W0, fe_b0.reshape(1, -1), fe_W1,
      fe_b1.reshape(1, -1), fn_W0, fn_b0.reshape(1, -1), fn_W1,
      fn_b1.reshape(1, -1))

    return out4.reshape(B, N, FO)


# speedup vs baseline: 1.3362x; 1.3362x over previous
import jax
import jax.numpy as jnp
from jax.experimental import pallas as pl
from jax.experimental.pallas import tpu as pltpu

_ALPHA = 0.2
_PACK = 4
_BB = 16


def _lrelu(v):
    return jnp.maximum(v, _ALPHA * v)


def _mp_kernel(x_ref, x4_ref, feW0_ref, feb0_ref, feW1_ref, feb1_ref,
               fnW0_ref, fnb0_ref, fnW1_ref, fnb1_ref, out_ref,
               w0a4_s, w1bd_s, p4_s, na4_s, nb4_s, nw14_s):
    k = _PACK
    n, d = x_ref.shape[1], x_ref.shape[2]
    m = n // k
    f1 = feW1_ref.shape[0]
    f2 = feW1_ref.shape[1]
    fn = fnW0_ref.shape[1]
    fo = fnW1_ref.shape[1]

    @pl.when(pl.program_id(0) == 0)
    def _prep():
        w0a4_s[...] = jnp.zeros_like(w0a4_s)
        w1bd_s[...] = jnp.zeros_like(w1bd_s)
        p4_s[...] = jnp.zeros_like(p4_s)
        na4_s[...] = jnp.zeros_like(na4_s)
        nb4_s[...] = jnp.zeros_like(nb4_s)
        nw14_s[...] = jnp.zeros_like(nw14_s)
        for i in range(k):
            w0a4_s[i * d:(i + 1) * d, i * f1:(i + 1) * f1] = feW0_ref[0:d]
            w1bd_s[i * f1:(i + 1) * f1, i * f2:(i + 1) * f2] = feW1_ref[...]
            p4_s[i:i + 1, i * f1:(i + 1) * f1] = feW0_ref[2 * d:2 * d + 1]
            na4_s[i * f2:(i + 1) * f2, i * fn:(i + 1) * fn] = fnW0_ref[0:f2]
            nb4_s[i * d:(i + 1) * d, i * fn:(i + 1) * fn] = fnW0_ref[f2:]
            nw14_s[i * fn:(i + 1) * fn, i * fo:(i + 1) * fo] = fnW1_ref[...]

    w0b = feW0_ref[d:2 * d]
    b04 = jnp.tile(feb0_ref[...], (1, k))
    b14 = jnp.tile(feb1_ref[...], (1, k))
    nb04 = jnp.tile(fnb0_ref[...], (1, k))
    nb14 = jnp.tile(fnb1_ref[...], (1, k))

    for s in range(x_ref.shape[0]):
        x = x_ref[s]

        u4 = jnp.dot(x4_ref[s], w0a4_s[...],
                     preferred_element_type=jnp.float32)
        v = jnp.dot(x, w0b, preferred_element_type=jnp.float32)
        v4 = jnp.tile(v, (1, k)) + b04

        xx = x * x
        g = jax.lax.dot_general(x, x, (((1,), (1,)), ((), ())),
                                preferred_element_type=jnp.float32)
        sq_col = jnp.sum(xx, axis=1, keepdims=True)
        ones_row = jnp.ones((1, d), jnp.float32)
        sq_row = jax.lax.dot_general(ones_row, xx, (((1,), (1,)), ((), ())),
                                     preferred_element_type=jnp.float32)
        dist = jnp.sqrt(jnp.maximum(sq_col + sq_row - 2.0 * g, 0.0))

        dist4 = jnp.transpose(dist.reshape(m, k, n),
                              (0, 2, 1)).reshape(m * n, k)
        dterm = jnp.dot(dist4, p4_s[...],
                        preferred_element_type=jnp.float32)

        e = u4[:, None, :] + v4[None, :, :] + dterm.reshape(m, n, k * d)
        e = _lrelu(e)

        e2 = jnp.dot(e.reshape(m * n, k * d), w1bd_s[...],
                     preferred_element_type=jnp.float32) + b14
        e2 = _lrelu(e2)

        agg4 = jnp.sum(e2.reshape(m, n, -1), axis=1)

        h = jnp.dot(agg4, na4_s[...], preferred_element_type=jnp.float32)
        h = h + jnp.dot(x4_ref[s], nb4_s[...],
                        preferred_element_type=jnp.float32)
        h = _lrelu(h + nb04)
        out_ref[s] = jnp.dot(h, nw14_s[...],
                             preferred_element_type=jnp.float32) + nb14


def kernel(x, fe_W0, fe_b0, fe_W1, fe_b1, fn_W0, fn_b0, fn_W1, fn_b1):
    B, N, D = x.shape
    F1 = fe_W0.shape[1]
    F2 = fe_W1.shape[1]
    FN = fn_W0.shape[1]
    FO = fn_W1.shape[1]
    k = _PACK
    m = N // k
    bb = _BB

    full = lambda shape: pl.BlockSpec(shape, lambda b: (0,) * len(shape))

    out4 = pl.pallas_call(
        _mp_kernel,
        grid=(B // bb,),
        in_specs=[
            pl.BlockSpec((bb, N, D), lambda b: (b, 0, 0)),
            pl.BlockSpec((bb, m, k * D), lambda b: (b, 0, 0)),
            full(fe_W0.shape), full((1, F1)), full(fe_W1.shape), full((1, F2)),
            full(fn_W0.shape), full((1, FN)), full(fn_W1.shape), full((1, FO)),
        ],
        out_specs=pl.BlockSpec((bb, m, k * FO), lambda b: (b, 0, 0)),
        out_shape=jax.ShapeDtypeStruct((B, m, k * FO), jnp.float32),
        scratch_shapes=[
            pltpu.VMEM((k * D, k * F1), jnp.float32),
            pltpu.VMEM((k * F1, k * F2), jnp.float32),
            pltpu.VMEM((k, k * F1), jnp.float32),
            pltpu.VMEM((k * F2, k * FN), jnp.float32),
            pltpu.VMEM((k * D, k * FN), jnp.float32),
            pltpu.VMEM((k * FN, k * FO), jnp.float32),
        ],
    )(x, x.reshape(B, m, k * D), fe_W0, fe_b0.reshape(1, -1), fe_W1,
      fe_b1.reshape(1, -1), fn_W0, fn_b0.reshape(1, -1), fn_W1,
      fn_b1.reshape(1, -1))

    return out4.reshape(B, N, FO)
